# Initial kernel scaffold; baseline (speedup 1.0000x reference)
import jax
import jax.numpy as jnp
from jax.experimental import pallas as pl


def kernel(sc_data, st_x, edge_index, W_fe, b_fe, W_e, b_e, W_fd, b_fd, W_d, b_d):
    raise NotImplementedError("write your pallas kernel here")


# speedup vs baseline: 9.4652x; 9.4652x over previous
import functools

import jax
import jax.numpy as jnp
from jax import lax
from jax.experimental import pallas as pl
from jax.experimental.pallas import tpu as pltpu
from jax.experimental.pallas import tpu_sc as plsc

N_SC_NODES = 20000
N_ST_NODES = 10000
N_EDGES = 320000
D_IN = 128
D_HID = 512
D_EMB = 16

NC = 2
NS = 16
NW = NC * NS
W1 = 144
EPT = N_EDGES // NW
CH = 80
NCHUNK = EPT // CH
RPT = N_ST_NODES // NS

_MM = dict(preferred_element_type=jnp.float32, precision=lax.Precision.HIGHEST)


def _elu(x):
    return jnp.where(x > 0, x, jnp.exp(jnp.minimum(x, 0.0)) - 1.0)


@functools.lru_cache(maxsize=None)
def _make_sc_agg(width):
    mesh = plsc.VectorSubcoreMesh(
        core_axis_name="c", subcore_axis_name="s", num_cores=NC, num_subcores=NS
    )

    @functools.partial(
        pl.kernel,
        out_type=jax.ShapeDtypeStruct((NC, N_ST_NODES, width), jnp.float32),
        mesh=mesh,
        scratch_types=[
            pltpu.VMEM((CH,), jnp.int32),
            pltpu.VMEM((CH,), jnp.int32),
            pltpu.VMEM((CH, width), jnp.float32),
            pltpu.VMEM_SHARED((N_ST_NODES, width), jnp.float32),
            pltpu.SemaphoreType.DMA,
        ],
        compiler_params=pltpu.CompilerParams(use_tc_tiling_on_sc=False),
    )
    def agg_kernel(table, src, dst, zeros, out, sidx, didx, rows, shared, sem):
        c = lax.axis_index("c")
        s = lax.axis_index("s")
        worker = c * NS + s
        r0 = s * RPT
        pltpu.sync_copy(zeros.at[pl.ds(r0, RPT)], shared.at[pl.ds(r0, RPT)])
        plsc.subcore_barrier()

        ebase = worker * EPT

        def chunk(j, carry):
            b = ebase + j * CH
            pltpu.sync_copy(src.at[pl.ds(b, CH)], sidx)
            pltpu.sync_copy(dst.at[pl.ds(b, CH)], didx)
            pltpu.async_copy(table.at[sidx], rows, sem).wait()
            pltpu.sync_copy(rows, shared.at[didx], add=True)
            return carry

        lax.fori_loop(0, NCHUNK, chunk, 0)
        plsc.subcore_barrier()
        pltpu.sync_copy(shared.at[pl.ds(r0, RPT)], out.at[c, pl.ds(r0, RPT)])

    return agg_kernel


def _tc_sc_branch(x, wfe, bfe, we, be, wfd, bfd, wd, bd):
    R = 400
    grid = (N_SC_NODES // R,)

    def body(x_r, wfe_r, bfe_r, we_r, be_r, wfd_r, bfd_r, wd_r, bd_r, emb_r, rec_r):
        h = jnp.dot(x_r[...], wfe_r[...], **_MM) + bfe_r[...]
        emb = jnp.dot(_elu(h), we_r[...], **_MM) + be_r[...]
        emb_r[...] = emb
        rh = jnp.dot(emb, wfd_r[...], **_MM) + bfd_r[...]
        rec_r[...] = jnp.dot(_elu(rh), wd_r[...], **_MM) + bd_r[...]

    full = lambda shape: pl.BlockSpec(shape, lambda i: (0, 0))
    return pl.pallas_call(
        body,
        grid=grid,
        in_specs=[
            pl.BlockSpec((R, D_IN), lambda i: (i, 0)),
            full((D_IN, D_HID)), full((1, D_HID)),
            full((D_HID, D_EMB)), full((1, D_EMB)),
            full((D_EMB, D_HID)), full((1, D_HID)),
            full((D_HID, D_IN)), full((1, D_IN)),
        ],
        out_specs=[
            pl.BlockSpec((R, D_EMB), lambda i: (i, 0)),
            pl.BlockSpec((R, D_IN), lambda i: (i, 0)),
        ],
        out_shape=[
            jax.ShapeDtypeStruct((N_SC_NODES, D_EMB), jnp.float32),
            jax.ShapeDtypeStruct((N_SC_NODES, D_IN), jnp.float32),
        ],
    )(x, wfe, bfe, we, be, wfd, bfd, wd, bd)


def _tc_st_encode(agg1, wfe, bfe, we, be):
    R = 400
    grid = (N_ST_NODES // R,)

    def body(agg_r, wfe_r, bfe_r, we_r, be_r, emb_r):
        a = agg_r[0] + agg_r[1]
        deg = a[:, D_IN:D_IN + 1]
        x = a[:, :D_IN]
        nx = x / jnp.maximum(deg, 1.0)
        m = jnp.minimum(deg, 1.0)
        h = jnp.dot(nx, wfe_r[...], **_MM) + m * bfe_r[...]
        emb_r[...] = jnp.dot(_elu(h), we_r[...], **_MM) + be_r[...]

    full = lambda shape: pl.BlockSpec(shape, lambda i: (0, 0))
    return pl.pallas_call(
        body,
        grid=grid,
        in_specs=[
            pl.BlockSpec((NC, R, W1), lambda i: (0, i, 0)),
            pl.BlockSpec((D_IN, D_HID), lambda i: (0, 0)), full((1, D_HID)),
            pl.BlockSpec((D_HID, D_EMB), lambda i: (0, 0)), full((1, D_EMB)),
        ],
        out_specs=pl.BlockSpec((R, D_EMB), lambda i: (i, 0)),
        out_shape=jax.ShapeDtypeStruct((N_ST_NODES, D_EMB), jnp.float32),
    )(agg1, wfe, bfe, we, be)


def _tc_st_decode(agg2, agg1, wfd, bfd, wd, bd):
    R = 400
    grid = (N_ST_NODES // R,)

    def body(agg2_r, agg1_r, wfd_r, bfd_r, wd_r, bd_r, rec_r):
        a2 = agg2_r[0] + agg2_r[1]
        deg = agg1_r[0, :, D_IN:D_IN + 1] + agg1_r[1, :, D_IN:D_IN + 1]
        nx = a2 / jnp.maximum(deg, 1.0)
        m = jnp.minimum(deg, 1.0)
        rh = jnp.dot(nx, wfd_r[...], **_MM) + m * bfd_r[...]
        rec_r[...] = jnp.dot(_elu(rh), wd_r[...], **_MM) + bd_r[...]

    full = lambda shape: pl.BlockSpec(shape, lambda i: (0, 0))
    return pl.pallas_call(
        body,
        grid=grid,
        in_specs=[
            pl.BlockSpec((NC, R, D_EMB), lambda i: (0, i, 0)),
            pl.BlockSpec((NC, R, W1), lambda i: (0, i, 0)),
            pl.BlockSpec((D_EMB, D_HID), lambda i: (0, 0)), full((1, D_HID)),
            pl.BlockSpec((D_HID, D_IN), lambda i: (0, 0)), full((1, D_IN)),
        ],
        out_specs=pl.BlockSpec((R, D_IN), lambda i: (i, 0)),
        out_shape=jax.ShapeDtypeStruct((N_ST_NODES, D_IN), jnp.float32),
    )(agg2, agg1, wfd, bfd, wd, bd)


def kernel(sc_data, st_x, edge_index, W_fe, b_fe, W_e, b_e, W_fd, b_fd, W_d, b_d):
    src = edge_index[0]
    dst = edge_index[1]

    ones_col = jnp.ones((N_ST_NODES, 1), jnp.float32)
    pad = jnp.zeros((N_ST_NODES, W1 - D_IN - 1), jnp.float32)
    table1 = jnp.concatenate([st_x, ones_col, pad], axis=1)
    z1 = jnp.zeros((N_ST_NODES, W1), jnp.float32)
    z2 = jnp.zeros((N_ST_NODES, D_EMB), jnp.float32)

    bfe = b_fe.reshape(1, D_HID)
    be = b_e.reshape(1, D_EMB)
    bfd = b_fd.reshape(1, D_HID)
    bd = b_d.reshape(1, D_IN)

    agg1 = _make_sc_agg(W1)(table1, src, dst, z1)
    sc_emb, sc_rec = _tc_sc_branch(
        sc_data, W_fe, bfe, W_e, be, W_fd, bfd, W_d, bd)
    st_emb = _tc_st_encode(agg1, W_fe, bfe, W_e, be)
    agg2 = _make_sc_agg(D_EMB)(st_emb, src, dst, z2)
    st_rec = _tc_st_decode(agg2, agg1, W_fd, bfd, W_d, bd)
    return (sc_emb, st_emb, sc_rec, st_rec)


# speedup vs baseline: 18.3224x; 1.9358x over previous
import functools

import jax
import jax.numpy as jnp
from jax import lax
from jax.experimental import pallas as pl
from jax.experimental.pallas import tpu as pltpu
from jax.experimental.pallas import tpu_sc as plsc

N_SC_NODES = 20000
N_ST_NODES = 10000
N_EDGES = 320000
D_IN = 128
D_HID = 512
D_EMB = 16

NC = 2
NS = 16
NW = NC * NS
W1 = 144
EPT = N_EDGES // NW
CH = 80
NCHUNK = EPT // CH
RPT = N_ST_NODES // NS

_MM = dict(preferred_element_type=jnp.float32)


def _elu(x):
    return jnp.where(x > 0, x, jnp.exp(jnp.minimum(x, 0.0)) - 1.0)


@functools.lru_cache(maxsize=None)
def _make_sc_agg(width):
    mesh = plsc.VectorSubcoreMesh(
        core_axis_name="c", subcore_axis_name="s", num_cores=NC, num_subcores=NS
    )

    @functools.partial(
        pl.kernel,
        out_type=jax.ShapeDtypeStruct((NC, N_ST_NODES, width), jnp.float32),
        mesh=mesh,
        scratch_types=[
            pltpu.VMEM((CH,), jnp.int32),
            pltpu.VMEM((CH,), jnp.int32),
            pltpu.VMEM((CH,), jnp.int32),
            pltpu.VMEM((CH,), jnp.int32),
            pltpu.VMEM((CH, width), jnp.float32),
            pltpu.VMEM((CH, width), jnp.float32),
            pltpu.VMEM_SHARED((N_ST_NODES, width), jnp.float32),
            pltpu.SemaphoreType.DMA,
            pltpu.SemaphoreType.DMA,
            pltpu.SemaphoreType.DMA,
            pltpu.SemaphoreType.DMA,
            pltpu.SemaphoreType.DMA,
            pltpu.SemaphoreType.DMA,
        ],
        compiler_params=pltpu.CompilerParams(use_tc_tiling_on_sc=False),
    )
    def agg_kernel(table, src, dst, zeros, out,
                   s0, s1, d0, d1, r0b, r1b, shared,
                   ssi0, ssi1, sdi0, sdi1, sg0, sg1):
        c = lax.axis_index("c")
        s = lax.axis_index("s")
        worker = c * NS + s
        rr0 = s * RPT
        pltpu.sync_copy(zeros.at[pl.ds(rr0, RPT)], shared.at[pl.ds(rr0, RPT)])
        plsc.subcore_barrier()

        ebase = worker * EPT
        S = (s0, s1)
        D = (d0, d1)
        R = (r0b, r1b)
        SSI = (ssi0, ssi1)
        SDI = (sdi0, sdi1)
        SG = (sg0, sg1)

        def eslice(j):
            return pl.ds(ebase + j * CH, CH)

        pltpu.async_copy(src.at[eslice(0)], s0, ssi0)
        pltpu.async_copy(dst.at[eslice(0)], d0, sdi0)
        pltpu.async_copy(src.at[eslice(1)], s1, ssi1)
        pltpu.async_copy(dst.at[eslice(1)], d1, sdi1)
        pltpu.make_async_copy(src.at[eslice(0)], s0, ssi0).wait()
        pltpu.make_async_copy(dst.at[eslice(0)], d0, sdi0).wait()
        pltpu.async_copy(table.at[s0], r0b, sg0)

        def halfstep(k, p):
            q = 1 - p

            @pl.when(k + 1 < NCHUNK)
            def _():
                pltpu.make_async_copy(src.at[eslice(k + 1)], S[q], SSI[q]).wait()
                pltpu.make_async_copy(dst.at[eslice(k + 1)], D[q], SDI[q]).wait()
                pltpu.async_copy(table.at[S[q]], R[q], SG[q])

            @pl.when(k < NCHUNK)
            def _():
                pltpu.make_async_copy(table.at[S[p]], R[p], SG[p]).wait()
                pltpu.sync_copy(R[p], shared.at[D[p]], add=True)

            @pl.when(k + 2 < NCHUNK)
            def _():
                pltpu.async_copy(src.at[eslice(k + 2)], S[p], SSI[p])
                pltpu.async_copy(dst.at[eslice(k + 2)], D[p], SDI[p])

        def pair(t, carry):
            halfstep(2 * t, 0)
            halfstep(2 * t + 1, 1)
            return carry

        lax.fori_loop(0, (NCHUNK + 1) // 2, pair, 0)
        plsc.subcore_barrier()
        pltpu.sync_copy(shared.at[pl.ds(rr0, RPT)], out.at[c, pl.ds(rr0, RPT)])

    return agg_kernel


def _tc_sc_branch(x, wfe, bfe, we, be, wfd, bfd, wd, bd):
    R = 400
    grid = (N_SC_NODES // R,)

    def body(x_r, wfe_r, bfe_r, we_r, be_r, wfd_r, bfd_r, wd_r, bd_r, emb_r, rec_r):
        h = jnp.dot(x_r[...], wfe_r[...], **_MM) + bfe_r[...]
        emb = jnp.dot(_elu(h), we_r[...], **_MM) + be_r[...]
        emb_r[...] = emb
        rh = jnp.dot(emb, wfd_r[...], **_MM) + bfd_r[...]
        rec_r[...] = jnp.dot(_elu(rh), wd_r[...], **_MM) + bd_r[...]

    full = lambda shape: pl.BlockSpec(shape, lambda i: (0, 0))
    return pl.pallas_call(
        body,
        grid=grid,
        in_specs=[
            pl.BlockSpec((R, D_IN), lambda i: (i, 0)),
            full((D_IN, D_HID)), full((1, D_HID)),
            full((D_HID, D_EMB)), full((1, D_EMB)),
            full((D_EMB, D_HID)), full((1, D_HID)),
            full((D_HID, D_IN)), full((1, D_IN)),
        ],
        out_specs=[
            pl.BlockSpec((R, D_EMB), lambda i: (i, 0)),
            pl.BlockSpec((R, D_IN), lambda i: (i, 0)),
        ],
        out_shape=[
            jax.ShapeDtypeStruct((N_SC_NODES, D_EMB), jnp.float32),
            jax.ShapeDtypeStruct((N_SC_NODES, D_IN), jnp.float32),
        ],
    )(x, wfe, bfe, we, be, wfd, bfd, wd, bd)


def _tc_st_encode(agg1, wfe, bfe, we, be):
    R = 400
    grid = (N_ST_NODES // R,)

    def body(agg_r, wfe_r, bfe_r, we_r, be_r, emb_r):
        a = agg_r[0] + agg_r[1]
        deg = a[:, D_IN:D_IN + 1]
        x = a[:, :D_IN]
        nx = x / jnp.maximum(deg, 1.0)
        m = jnp.minimum(deg, 1.0)
        h = jnp.dot(nx, wfe_r[...], **_MM) + m * bfe_r[...]
        emb_r[...] = jnp.dot(_elu(h), we_r[...], **_MM) + be_r[...]

    full = lambda shape: pl.BlockSpec(shape, lambda i: (0, 0))
    return pl.pallas_call(
        body,
        grid=grid,
        in_specs=[
            pl.BlockSpec((NC, R, W1), lambda i: (0, i, 0)),
            pl.BlockSpec((D_IN, D_HID), lambda i: (0, 0)), full((1, D_HID)),
            pl.BlockSpec((D_HID, D_EMB), lambda i: (0, 0)), full((1, D_EMB)),
        ],
        out_specs=pl.BlockSpec((R, D_EMB), lambda i: (i, 0)),
        out_shape=jax.ShapeDtypeStruct((N_ST_NODES, D_EMB), jnp.float32),
    )(agg1, wfe, bfe, we, be)


def _tc_st_decode(agg2, agg1, wfd, bfd, wd, bd):
    R = 400
    grid = (N_ST_NODES // R,)

    def body(agg2_r, agg1_r, wfd_r, bfd_r, wd_r, bd_r, rec_r):
        a2 = agg2_r[0] + agg2_r[1]
        deg = agg1_r[0, :, D_IN:D_IN + 1] + agg1_r[1, :, D_IN:D_IN + 1]
        nx = a2 / jnp.maximum(deg, 1.0)
        m = jnp.minimum(deg, 1.0)
        rh = jnp.dot(nx, wfd_r[...], **_MM) + m * bfd_r[...]
        rec_r[...] = jnp.dot(_elu(rh), wd_r[...], **_MM) + bd_r[...]

    full = lambda shape: pl.BlockSpec(shape, lambda i: (0, 0))
    return pl.pallas_call(
        body,
        grid=grid,
        in_specs=[
            pl.BlockSpec((NC, R, D_EMB), lambda i: (0, i, 0)),
            pl.BlockSpec((NC, R, W1), lambda i: (0, i, 0)),
            pl.BlockSpec((D_EMB, D_HID), lambda i: (0, 0)), full((1, D_HID)),
            pl.BlockSpec((D_HID, D_IN), lambda i: (0, 0)), full((1, D_IN)),
        ],
        out_specs=pl.BlockSpec((R, D_IN), lambda i: (i, 0)),
        out_shape=jax.ShapeDtypeStruct((N_ST_NODES, D_IN), jnp.float32),
    )(agg2, agg1, wfd, bfd, wd, bd)


def kernel(sc_data, st_x, edge_index, W_fe, b_fe, W_e, b_e, W_fd, b_fd, W_d, b_d):
    src = edge_index[0]
    dst = edge_index[1]

    ones_col = jnp.ones((N_ST_NODES, 1), jnp.float32)
    pad = jnp.zeros((N_ST_NODES, W1 - D_IN - 1), jnp.float32)
    table1 = jnp.concatenate([st_x, ones_col, pad], axis=1)
    z1 = jnp.zeros((N_ST_NODES, W1), jnp.float32)
    z2 = jnp.zeros((N_ST_NODES, D_EMB), jnp.float32)

    bfe = b_fe.reshape(1, D_HID)
    be = b_e.reshape(1, D_EMB)
    bfd = b_fd.reshape(1, D_HID)
    bd = b_d.reshape(1, D_IN)

    agg1 = _make_sc_agg(W1)(table1, src, dst, z1)
    sc_emb, sc_rec = _tc_sc_branch(
        sc_data, W_fe, bfe, W_e, be, W_fd, bfd, W_d, bd)
    st_emb = _tc_st_encode(agg1, W_fe, bfe, W_e, be)
    agg2 = _make_sc_agg(D_EMB)(st_emb, src, dst, z2)
    st_rec = _tc_st_decode(agg2, agg1, W_fd, bfd, W_d, bd)
    return (sc_emb, st_emb, sc_rec, st_rec)


# speedup vs baseline: 19.8707x; 1.0845x over previous
import functools

import jax
import jax.numpy as jnp
from jax import lax
from jax.experimental import pallas as pl
from jax.experimental.pallas import tpu as pltpu
from jax.experimental.pallas import tpu_sc as plsc

N_SC_NODES = 20000
N_ST_NODES = 10000
N_EDGES = 320000
D_IN = 128
D_HID = 512
D_EMB = 16

NC = 2
NS = 16
NW = NC * NS
W1 = 144
EPT = N_EDGES // NW
CH = 80
NCHUNK = EPT // CH
TAIL = EPT - NCHUNK * CH
NSLOT = 3
RPT = N_ST_NODES // NS

_MM = dict(preferred_element_type=jnp.float32)


def _elu(x):
    return jnp.where(x > 0, x, jnp.exp(jnp.minimum(x, 0.0)) - 1.0)


@functools.lru_cache(maxsize=None)
def _make_sc_agg(width):
    mesh = plsc.VectorSubcoreMesh(
        core_axis_name="c", subcore_axis_name="s", num_cores=NC, num_subcores=NS
    )

    @functools.partial(
        pl.kernel,
        out_type=jax.ShapeDtypeStruct((NC, N_ST_NODES, width), jnp.float32),
        mesh=mesh,
        scratch_types=(
            [pltpu.VMEM((CH,), jnp.int32) for _ in range(NSLOT)]
            + [pltpu.VMEM((CH,), jnp.int32) for _ in range(NSLOT)]
            + [pltpu.VMEM((CH, width), jnp.float32) for _ in range(NSLOT)]
            + [pltpu.VMEM_SHARED((N_ST_NODES, width), jnp.float32)]
            + [pltpu.SemaphoreType.DMA for _ in range(4 * NSLOT)]
        ),
        compiler_params=pltpu.CompilerParams(use_tc_tiling_on_sc=False),
    )
    def agg_kernel(table, src, dst, zeros, out,
                   s0, s1, s2, d0, d1, d2, r0b, r1b, r2b, shared,
                   si0, si1, si2, di0, di1, di2,
                   g0, g1, g2, ss0, ss1, ss2):
        c = lax.axis_index("c")
        s = lax.axis_index("s")
        worker = c * NS + s
        rr0 = s * RPT
        pltpu.sync_copy(zeros.at[pl.ds(rr0, RPT)], shared.at[pl.ds(rr0, RPT)])
        plsc.subcore_barrier()

        ebase = worker * EPT
        S = (s0, s1, s2)
        D = (d0, d1, d2)
        R = (r0b, r1b, r2b)
        SI = (si0, si1, si2)
        DI = (di0, di1, di2)
        G = (g0, g1, g2)
        SS = (ss0, ss1, ss2)

        def eslice(j):
            return pl.ds(ebase + j * CH, CH)

        def issue_idx(j, h):
            pltpu.async_copy(src.at[eslice(j)], S[h], SI[h])
            pltpu.async_copy(dst.at[eslice(j)], D[h], DI[h])

        def wait_idx(j, h):
            pltpu.make_async_copy(src.at[eslice(j)], S[h], SI[h]).wait()
            pltpu.make_async_copy(dst.at[eslice(j)], D[h], DI[h]).wait()

        issue_idx(0, 0)
        issue_idx(1, 1)
        wait_idx(0, 0)
        pltpu.async_copy(table.at[s0], r0b, g0)

        def halfstep(k, h):
            h1 = (h + 1) % NSLOT
            h2 = (h + 2) % NSLOT

            @pl.when(k + 1 < NCHUNK)
            def _():
                wait_idx(k + 1, h1)
                pltpu.async_copy(table.at[S[h1]], R[h1], G[h1])

            @pl.when((k >= 1) & (k < NCHUNK + 1))
            def _():
                pltpu.make_async_copy(R[h2], shared.at[D[h2]], SS[h2]).wait()

            @pl.when(k < NCHUNK)
            def _():
                pltpu.make_async_copy(table.at[S[h]], R[h], G[h]).wait()
                pltpu.async_copy(R[h], shared.at[D[h]], SS[h], add=True)

            @pl.when(k + 2 < NCHUNK)
            def _():
                issue_idx(k + 2, h2)

        def step(t, carry):
            for hh in range(NSLOT):
                halfstep(NSLOT * t + hh, hh)
            return carry

        lax.fori_loop(0, (NCHUNK + 1) // NSLOT, step, 0)

        plsc.subcore_barrier()
        pltpu.sync_copy(shared.at[pl.ds(rr0, RPT)], out.at[c, pl.ds(rr0, RPT)])

    return agg_kernel


def _tc_sc_branch(x, wfe, bfe, we, be, wfd, bfd, wd, bd):
    R = 400
    grid = (N_SC_NODES // R,)

    def body(x_r, wfe_r, bfe_r, we_r, be_r, wfd_r, bfd_r, wd_r, bd_r, emb_r, rec_r):
        h = jnp.dot(x_r[...], wfe_r[...], **_MM) + bfe_r[...]
        emb = jnp.dot(_elu(h), we_r[...], **_MM) + be_r[...]
        emb_r[...] = emb
        rh = jnp.dot(emb, wfd_r[...], **_MM) + bfd_r[...]
        rec_r[...] = jnp.dot(_elu(rh), wd_r[...], **_MM) + bd_r[...]

    full = lambda shape: pl.BlockSpec(shape, lambda i: (0, 0))
    return pl.pallas_call(
        body,
        grid=grid,
        in_specs=[
            pl.BlockSpec((R, D_IN), lambda i: (i, 0)),
            full((D_IN, D_HID)), full((1, D_HID)),
            full((D_HID, D_EMB)), full((1, D_EMB)),
            full((D_EMB, D_HID)), full((1, D_HID)),
            full((D_HID, D_IN)), full((1, D_IN)),
        ],
        out_specs=[
            pl.BlockSpec((R, D_EMB), lambda i: (i, 0)),
            pl.BlockSpec((R, D_IN), lambda i: (i, 0)),
        ],
        out_shape=[
            jax.ShapeDtypeStruct((N_SC_NODES, D_EMB), jnp.float32),
            jax.ShapeDtypeStruct((N_SC_NODES, D_IN), jnp.float32),
        ],
    )(x, wfe, bfe, we, be, wfd, bfd, wd, bd)


def _tc_st_encode(agg1, wfe, bfe, we, be):
    R = 400
    grid = (N_ST_NODES // R,)

    def body(agg_r, wfe_r, bfe_r, we_r, be_r, emb_r):
        a = agg_r[0] + agg_r[1]
        deg = a[:, D_IN:D_IN + 1]
        x = a[:, :D_IN]
        nx = x / jnp.maximum(deg, 1.0)
        m = jnp.minimum(deg, 1.0)
        h = jnp.dot(nx, wfe_r[...], **_MM) + m * bfe_r[...]
        emb_r[...] = jnp.dot(_elu(h), we_r[...], **_MM) + be_r[...]

    full = lambda shape: pl.BlockSpec(shape, lambda i: (0, 0))
    return pl.pallas_call(
        body,
        grid=grid,
        in_specs=[
            pl.BlockSpec((NC, R, W1), lambda i: (0, i, 0)),
            pl.BlockSpec((D_IN, D_HID), lambda i: (0, 0)), full((1, D_HID)),
            pl.BlockSpec((D_HID, D_EMB), lambda i: (0, 0)), full((1, D_EMB)),
        ],
        out_specs=pl.BlockSpec((R, D_EMB), lambda i: (i, 0)),
        out_shape=jax.ShapeDtypeStruct((N_ST_NODES, D_EMB), jnp.float32),
    )(agg1, wfe, bfe, we, be)


def _tc_st_decode(agg2, agg1, wfd, bfd, wd, bd):
    R = 400
    grid = (N_ST_NODES // R,)

    def body(agg2_r, agg1_r, wfd_r, bfd_r, wd_r, bd_r, rec_r):
        a2 = agg2_r[0] + agg2_r[1]
        deg = agg1_r[0, :, D_IN:D_IN + 1] + agg1_r[1, :, D_IN:D_IN + 1]
        nx = a2 / jnp.maximum(deg, 1.0)
        m = jnp.minimum(deg, 1.0)
        rh = jnp.dot(nx, wfd_r[...], **_MM) + m * bfd_r[...]
        rec_r[...] = jnp.dot(_elu(rh), wd_r[...], **_MM) + bd_r[...]

    full = lambda shape: pl.BlockSpec(shape, lambda i: (0, 0))
    return pl.pallas_call(
        body,
        grid=grid,
        in_specs=[
            pl.BlockSpec((NC, R, D_EMB), lambda i: (0, i, 0)),
            pl.BlockSpec((NC, R, W1), lambda i: (0, i, 0)),
            pl.BlockSpec((D_EMB, D_HID), lambda i: (0, 0)), full((1, D_HID)),
            pl.BlockSpec((D_HID, D_IN), lambda i: (0, 0)), full((1, D_IN)),
        ],
        out_specs=pl.BlockSpec((R, D_IN), lambda i: (i, 0)),
        out_shape=jax.ShapeDtypeStruct((N_ST_NODES, D_IN), jnp.float32),
    )(agg2, agg1, wfd, bfd, wd, bd)


def kernel(sc_data, st_x, edge_index, W_fe, b_fe, W_e, b_e, W_fd, b_fd, W_d, b_d):
    src = edge_index[0]
    dst = edge_index[1]

    ones_col = jnp.ones((N_ST_NODES, 1), jnp.float32)
    pad = jnp.zeros((N_ST_NODES, W1 - D_IN - 1), jnp.float32)
    table1 = jnp.concatenate([st_x, ones_col, pad], axis=1)
    z1 = jnp.zeros((N_ST_NODES, W1), jnp.float32)
    z2 = jnp.zeros((N_ST_NODES, D_EMB), jnp.float32)

    bfe = b_fe.reshape(1, D_HID)
    be = b_e.reshape(1, D_EMB)
    bfd = b_fd.reshape(1, D_HID)
    bd = b_d.reshape(1, D_IN)

    agg1 = _make_sc_agg(W1)(table1, src, dst, z1)
    sc_emb, sc_rec = _tc_sc_branch(
        sc_data, W_fe, bfe, W_e, be, W_fd, bfd, W_d, bd)
    st_emb = _tc_st_encode(agg1, W_fe, bfe, W_e, be)
    agg2 = _make_sc_agg(D_EMB)(st_emb, src, dst, z2)
    st_rec = _tc_st_decode(agg2, agg1, W_fd, bfd, W_d, bd)
    return (sc_emb, st_emb, sc_rec, st_rec)


# speedup vs baseline: 20.7613x; 1.0448x over previous
import functools

import jax
import jax.numpy as jnp
from jax import lax
from jax.experimental import pallas as pl
from jax.experimental.pallas import tpu as pltpu
from jax.experimental.pallas import tpu_sc as plsc

N_SC_NODES = 20000
N_ST_NODES = 10000
N_EDGES = 320000
D_IN = 128
D_HID = 512
D_EMB = 16

NC = 2
NS = 16
NW = NC * NS
W1 = 144
EPT = N_EDGES // NW
RPT = N_ST_NODES // NS

_MM = dict(preferred_element_type=jnp.float32)


def _elu(x):
    return jnp.where(x > 0, x, jnp.exp(jnp.minimum(x, 0.0)) - 1.0)


@functools.lru_cache(maxsize=None)
def _make_sc_agg(width, ch, nslot, gd, idd, sd):
    nchunk = EPT // ch
    tail = EPT - nchunk * ch
    mesh = plsc.VectorSubcoreMesh(
        core_axis_name="c", subcore_axis_name="s", num_cores=NC, num_subcores=NS
    )

    @functools.partial(
        pl.kernel,
        out_type=jax.ShapeDtypeStruct((NC, N_ST_NODES, width), jnp.float32),
        mesh=mesh,
        scratch_types=(
            [pltpu.VMEM((ch,), jnp.int32) for _ in range(2 * nslot)]
            + [pltpu.VMEM((ch, width), jnp.float32) for _ in range(nslot)]
            + [pltpu.VMEM((tail or 8,), jnp.int32) for _ in range(2)]
            + [pltpu.VMEM((tail or 8, width), jnp.float32)]
            + [pltpu.VMEM_SHARED((N_ST_NODES, width), jnp.float32)]
            + [pltpu.SemaphoreType.DMA for _ in range(4 * nslot)]
        ),
        compiler_params=pltpu.CompilerParams(use_tc_tiling_on_sc=False),
    )
    def agg_kernel(table, src, dst, zeros, out, *scr):
        S = scr[0:nslot]
        D = scr[nslot:2 * nslot]
        R = scr[2 * nslot:3 * nslot]
        ts, td, tr = scr[3 * nslot:3 * nslot + 3]
        shared = scr[3 * nslot + 3]
        sems = scr[3 * nslot + 4:]
        SI = sems[0:nslot]
        DI = sems[nslot:2 * nslot]
        G = sems[2 * nslot:3 * nslot]
        SS = sems[3 * nslot:4 * nslot]

        c = lax.axis_index("c")
        s = lax.axis_index("s")
        worker = c * NS + s
        rr0 = s * RPT
        pltpu.sync_copy(zeros.at[pl.ds(rr0, RPT)], shared.at[pl.ds(rr0, RPT)])
        plsc.subcore_barrier()

        ebase = worker * EPT

        def eslice(j):
            return pl.ds(ebase + j * ch, ch)

        def issue_idx(j, h):
            pltpu.async_copy(src.at[eslice(j)], S[h], SI[h])
            pltpu.async_copy(dst.at[eslice(j)], D[h], DI[h])

        def wait_idx(j, h):
            pltpu.make_async_copy(src.at[eslice(j)], S[h], SI[h]).wait()
            pltpu.make_async_copy(dst.at[eslice(j)], D[h], DI[h]).wait()

        def issue_gather(j, h):
            wait_idx(j, h)
            pltpu.async_copy(table.at[S[h]], R[h], G[h])

        for j in range(idd):
            issue_idx(j, j % nslot)
        for j in range(gd):
            issue_gather(j, j % nslot)

        def halfstep(k, h):
            hg = (h + gd) % nslot
            hi = (h + idd) % nslot
            hs = (h - sd) % nslot

            @pl.when((k >= sd) & (k < nchunk + sd))
            def _():
                pltpu.make_async_copy(R[hs], shared.at[D[hs]], SS[hs]).wait()

            @pl.when(k + gd < nchunk)
            def _():
                issue_gather(k + gd, hg)

            @pl.when(k < nchunk)
            def _():
                pltpu.make_async_copy(table.at[S[h]], R[h], G[h]).wait()
                pltpu.async_copy(R[h], shared.at[D[h]], SS[h], add=True)

            @pl.when(k + idd < nchunk)
            def _():
                issue_idx(k + idd, hi)

        def step(t, carry):
            for hh in range(nslot):
                halfstep(nslot * t + hh, hh)
            return carry

        lax.fori_loop(0, (nchunk + sd + nslot - 1) // nslot, step, 0)

        if tail:
            tb = ebase + nchunk * ch
            pltpu.sync_copy(src.at[pl.ds(tb, tail)], ts)
            pltpu.sync_copy(dst.at[pl.ds(tb, tail)], td)
            pltpu.async_copy(table.at[ts], tr, G[0]).wait()
            pltpu.sync_copy(tr, shared.at[td], add=True)

        plsc.subcore_barrier()
        pltpu.sync_copy(shared.at[pl.ds(rr0, RPT)], out.at[c, pl.ds(rr0, RPT)])

    return agg_kernel


def _tc_sc_branch(x, wfe, bfe, we, be, wfd, bfd, wd, bd):
    R = 400
    grid = (N_SC_NODES // R,)

    def body(x_r, wfe_r, bfe_r, we_r, be_r, wfd_r, bfd_r, wd_r, bd_r, emb_r, rec_r):
        h = jnp.dot(x_r[...], wfe_r[...], **_MM) + bfe_r[...]
        emb = jnp.dot(_elu(h), we_r[...], **_MM) + be_r[...]
        emb_r[...] = emb
        rh = jnp.dot(emb, wfd_r[...], **_MM) + bfd_r[...]
        rec_r[...] = jnp.dot(_elu(rh), wd_r[...], **_MM) + bd_r[...]

    full = lambda shape: pl.BlockSpec(shape, lambda i: (0, 0))
    return pl.pallas_call(
        body,
        grid=grid,
        in_specs=[
            pl.BlockSpec((R, D_IN), lambda i: (i, 0)),
            full((D_IN, D_HID)), full((1, D_HID)),
            full((D_HID, D_EMB)), full((1, D_EMB)),
            full((D_EMB, D_HID)), full((1, D_HID)),
            full((D_HID, D_IN)), full((1, D_IN)),
        ],
        out_specs=[
            pl.BlockSpec((R, D_EMB), lambda i: (i, 0)),
            pl.BlockSpec((R, D_IN), lambda i: (i, 0)),
        ],
        out_shape=[
            jax.ShapeDtypeStruct((N_SC_NODES, D_EMB), jnp.float32),
            jax.ShapeDtypeStruct((N_SC_NODES, D_IN), jnp.float32),
        ],
    )(x, wfe, bfe, we, be, wfd, bfd, wd, bd)


def _tc_st_encode(agg1, wfe, bfe, we, be):
    R = 400
    grid = (N_ST_NODES // R,)

    def body(agg_r, wfe_r, bfe_r, we_r, be_r, emb_r):
        a = agg_r[0] + agg_r[1]
        deg = a[:, D_IN:D_IN + 1]
        x = a[:, :D_IN]
        nx = x / jnp.maximum(deg, 1.0)
        m = jnp.minimum(deg, 1.0)
        h = jnp.dot(nx, wfe_r[...], **_MM) + m * bfe_r[...]
        emb_r[...] = jnp.dot(_elu(h), we_r[...], **_MM) + be_r[...]

    full = lambda shape: pl.BlockSpec(shape, lambda i: (0, 0))
    return pl.pallas_call(
        body,
        grid=grid,
        in_specs=[
            pl.BlockSpec((NC, R, W1), lambda i: (0, i, 0)),
            pl.BlockSpec((D_IN, D_HID), lambda i: (0, 0)), full((1, D_HID)),
            pl.BlockSpec((D_HID, D_EMB), lambda i: (0, 0)), full((1, D_EMB)),
        ],
        out_specs=pl.BlockSpec((R, D_EMB), lambda i: (i, 0)),
        out_shape=jax.ShapeDtypeStruct((N_ST_NODES, D_EMB), jnp.float32),
    )(agg1, wfe, bfe, we, be)


def _tc_st_decode(agg2, agg1, wfd, bfd, wd, bd):
    R = 400
    grid = (N_ST_NODES // R,)

    def body(agg2_r, agg1_r, wfd_r, bfd_r, wd_r, bd_r, rec_r):
        a2 = agg2_r[0] + agg2_r[1]
        deg = agg1_r[0, :, D_IN:D_IN + 1] + agg1_r[1, :, D_IN:D_IN + 1]
        nx = a2 / jnp.maximum(deg, 1.0)
        m = jnp.minimum(deg, 1.0)
        rh = jnp.dot(nx, wfd_r[...], **_MM) + m * bfd_r[...]
        rec_r[...] = jnp.dot(_elu(rh), wd_r[...], **_MM) + bd_r[...]

    full = lambda shape: pl.BlockSpec(shape, lambda i: (0, 0))
    return pl.pallas_call(
        body,
        grid=grid,
        in_specs=[
            pl.BlockSpec((NC, R, D_EMB), lambda i: (0, i, 0)),
            pl.BlockSpec((NC, R, W1), lambda i: (0, i, 0)),
            pl.BlockSpec((D_EMB, D_HID), lambda i: (0, 0)), full((1, D_HID)),
            pl.BlockSpec((D_HID, D_IN), lambda i: (0, 0)), full((1, D_IN)),
        ],
        out_specs=pl.BlockSpec((R, D_IN), lambda i: (i, 0)),
        out_shape=jax.ShapeDtypeStruct((N_ST_NODES, D_IN), jnp.float32),
    )(agg2, agg1, wfd, bfd, wd, bd)


def kernel(sc_data, st_x, edge_index, W_fe, b_fe, W_e, b_e, W_fd, b_fd, W_d, b_d):
    src = edge_index[0]
    dst = edge_index[1]

    ones_col = jnp.ones((N_ST_NODES, 1), jnp.float32)
    pad = jnp.zeros((N_ST_NODES, W1 - D_IN - 1), jnp.float32)
    table1 = jnp.concatenate([st_x, ones_col, pad], axis=1)
    z1 = jnp.zeros((N_ST_NODES, W1), jnp.float32)
    z2 = jnp.zeros((N_ST_NODES, D_EMB), jnp.float32)

    bfe = b_fe.reshape(1, D_HID)
    be = b_e.reshape(1, D_EMB)
    bfd = b_fd.reshape(1, D_HID)
    bd = b_d.reshape(1, D_IN)

    agg1 = _make_sc_agg(W1, 64, 4, 1, 3, 1)(table1, src, dst, z1)
    sc_emb, sc_rec = _tc_sc_branch(
        sc_data, W_fe, bfe, W_e, be, W_fd, bfd, W_d, bd)
    st_emb = _tc_st_encode(agg1, W_fe, bfe, W_e, be)
    agg2 = _make_sc_agg(D_EMB, 80, 6, 2, 4, 2)(st_emb, src, dst, z2)
    st_rec = _tc_st_decode(agg2, agg1, W_fd, bfd, W_d, bd)
    return (sc_emb, st_emb, sc_rec, st_rec)


# speedup vs baseline: 21.7871x; 1.0494x over previous
import functools

import jax
import jax.numpy as jnp
from jax import lax
from jax.experimental import pallas as pl
from jax.experimental.pallas import tpu as pltpu
from jax.experimental.pallas import tpu_sc as plsc

N_SC_NODES = 20000
N_ST_NODES = 10000
N_EDGES = 320000
D_IN = 128
D_HID = 512
D_EMB = 16

NC = 2
NS = 16
NW = NC * NS
W1 = 144
EPT = N_EDGES // NW
RPT = N_ST_NODES // NS

_MM = dict(preferred_element_type=jnp.float32)


def _elu(x):
    return jnp.where(x > 0, x, jnp.exp(jnp.minimum(x, 0.0)) - 1.0)


@functools.lru_cache(maxsize=None)
def _make_sc_agg(width, ch, nslot, gd, idd, sd):
    nchunk = EPT // ch
    tail = EPT - nchunk * ch
    mesh = plsc.VectorSubcoreMesh(
        core_axis_name="c", subcore_axis_name="s", num_cores=NC, num_subcores=NS
    )

    @functools.partial(
        pl.kernel,
        out_type=jax.ShapeDtypeStruct((NC, N_ST_NODES, width), jnp.float32),
        mesh=mesh,
        scratch_types=(
            [pltpu.VMEM((ch,), jnp.int32) for _ in range(2 * nslot)]
            + [pltpu.VMEM((ch, width), jnp.float32) for _ in range(nslot)]
            + [pltpu.VMEM((tail or 8,), jnp.int32) for _ in range(2)]
            + [pltpu.VMEM((tail or 8, width), jnp.float32)]
            + [pltpu.VMEM_SHARED((N_ST_NODES, width), jnp.float32)]
            + [pltpu.SemaphoreType.DMA for _ in range(4 * nslot)]
        ),
        compiler_params=pltpu.CompilerParams(use_tc_tiling_on_sc=False),
    )
    def agg_kernel(table, edges, zeros, out, *scr):
        S = scr[0:nslot]
        D = scr[nslot:2 * nslot]
        R = scr[2 * nslot:3 * nslot]
        ts, td, tr = scr[3 * nslot:3 * nslot + 3]
        shared = scr[3 * nslot + 3]
        sems = scr[3 * nslot + 4:]
        SI = sems[0:nslot]
        DI = sems[nslot:2 * nslot]
        G = sems[2 * nslot:3 * nslot]
        SS = sems[3 * nslot:4 * nslot]

        c = lax.axis_index("c")
        s = lax.axis_index("s")
        worker = c * NS + s
        rr0 = s * RPT
        pltpu.sync_copy(zeros.at[pl.ds(rr0, RPT)], shared.at[pl.ds(rr0, RPT)])
        plsc.subcore_barrier()

        ebase = worker * EPT

        def eslice(j):
            return pl.ds(ebase + j * ch, ch)

        def issue_idx(j, h):
            pltpu.async_copy(edges.at[0, eslice(j)], S[h], SI[h])
            pltpu.async_copy(edges.at[1, eslice(j)], D[h], DI[h])

        def wait_idx(j, h):
            pltpu.make_async_copy(edges.at[0, eslice(j)], S[h], SI[h]).wait()
            pltpu.make_async_copy(edges.at[1, eslice(j)], D[h], DI[h]).wait()

        def issue_gather(j, h):
            wait_idx(j, h)
            pltpu.async_copy(table.at[S[h]], R[h], G[h])

        for j in range(idd):
            issue_idx(j, j % nslot)
        for j in range(gd):
            issue_gather(j, j % nslot)

        def halfstep(k, h):
            hg = (h + gd) % nslot
            hi = (h + idd) % nslot
            hs = (h - sd) % nslot

            @pl.when((k >= sd) & (k < nchunk + sd))
            def _():
                pltpu.make_async_copy(R[hs], shared.at[D[hs]], SS[hs]).wait()

            @pl.when(k + gd < nchunk)
            def _():
                issue_gather(k + gd, hg)

            @pl.when(k < nchunk)
            def _():
                pltpu.make_async_copy(table.at[S[h]], R[h], G[h]).wait()
                pltpu.async_copy(R[h], shared.at[D[h]], SS[h], add=True)

            @pl.when(k + idd < nchunk)
            def _():
                issue_idx(k + idd, hi)

        def step(t, carry):
            for hh in range(nslot):
                halfstep(nslot * t + hh, hh)
            return carry

        lax.fori_loop(0, (nchunk + sd + nslot - 1) // nslot, step, 0)

        if tail:
            tb = ebase + nchunk * ch
            pltpu.sync_copy(edges.at[0, pl.ds(tb, tail)], ts)
            pltpu.sync_copy(edges.at[1, pl.ds(tb, tail)], td)
            pltpu.async_copy(table.at[ts], tr, G[0]).wait()
            pltpu.sync_copy(tr, shared.at[td], add=True)

        plsc.subcore_barrier()
        pltpu.sync_copy(shared.at[pl.ds(rr0, RPT)], out.at[c, pl.ds(rr0, RPT)])

    return agg_kernel


def _tc_sc_branch(x, wfe, bfe, we, be, wfd, bfd, wd, bd):
    R = 400
    grid = (N_SC_NODES // R,)

    def body(x_r, wfe_r, bfe_r, we_r, be_r, wfd_r, bfd_r, wd_r, bd_r, emb_r, rec_r):
        h = jnp.dot(x_r[...], wfe_r[...], **_MM) + bfe_r[...]
        emb = jnp.dot(_elu(h), we_r[...], **_MM) + be_r[...]
        emb_r[...] = emb
        rh = jnp.dot(emb, wfd_r[...], **_MM) + bfd_r[...]
        rec_r[...] = jnp.dot(_elu(rh), wd_r[...], **_MM) + bd_r[...]

    full = lambda shape: pl.BlockSpec(shape, lambda i: (0, 0))
    return pl.pallas_call(
        body,
        grid=grid,
        in_specs=[
            pl.BlockSpec((R, D_IN), lambda i: (i, 0)),
            full((D_IN, D_HID)), full((1, D_HID)),
            full((D_HID, D_EMB)), full((1, D_EMB)),
            full((D_EMB, D_HID)), full((1, D_HID)),
            full((D_HID, D_IN)), full((1, D_IN)),
        ],
        out_specs=[
            pl.BlockSpec((R, D_EMB), lambda i: (i, 0)),
            pl.BlockSpec((R, D_IN), lambda i: (i, 0)),
        ],
        out_shape=[
            jax.ShapeDtypeStruct((N_SC_NODES, D_EMB), jnp.float32),
            jax.ShapeDtypeStruct((N_SC_NODES, D_IN), jnp.float32),
        ],
    )(x, wfe, bfe, we, be, wfd, bfd, wd, bd)


def _tc_st_encode(agg1, wfe, bfe, we, be):
    R = 400
    grid = (N_ST_NODES // R,)

    def body(agg_r, wfe_r, bfe_r, we_r, be_r, emb_r):
        a = agg_r[0] + agg_r[1]
        deg = a[:, D_IN:D_IN + 1]
        x = a[:, :D_IN]
        nx = x / jnp.maximum(deg, 1.0)
        m = jnp.minimum(deg, 1.0)
        h = jnp.dot(nx, wfe_r[...], **_MM) + m * bfe_r[...]
        emb_r[...] = jnp.dot(_elu(h), we_r[...], **_MM) + be_r[...]

    full = lambda shape: pl.BlockSpec(shape, lambda i: (0, 0))
    return pl.pallas_call(
        body,
        grid=grid,
        in_specs=[
            pl.BlockSpec((NC, R, W1), lambda i: (0, i, 0)),
            pl.BlockSpec((D_IN, D_HID), lambda i: (0, 0)), full((1, D_HID)),
            pl.BlockSpec((D_HID, D_EMB), lambda i: (0, 0)), full((1, D_EMB)),
        ],
        out_specs=pl.BlockSpec((R, D_EMB), lambda i: (i, 0)),
        out_shape=jax.ShapeDtypeStruct((N_ST_NODES, D_EMB), jnp.float32),
    )(agg1, wfe, bfe, we, be)


def _tc_st_decode(agg2, agg1, wfd, bfd, wd, bd):
    R = 400
    grid = (N_ST_NODES // R,)

    def body(agg2_r, agg1_r, wfd_r, bfd_r, wd_r, bd_r, rec_r):
        a2 = agg2_r[0] + agg2_r[1]
        deg = agg1_r[0, :, D_IN:D_IN + 1] + agg1_r[1, :, D_IN:D_IN + 1]
        nx = a2 / jnp.maximum(deg, 1.0)
        m = jnp.minimum(deg, 1.0)
        rh = jnp.dot(nx, wfd_r[...], **_MM) + m * bfd_r[...]
        rec_r[...] = jnp.dot(_elu(rh), wd_r[...], **_MM) + bd_r[...]

    full = lambda shape: pl.BlockSpec(shape, lambda i: (0, 0))
    return pl.pallas_call(
        body,
        grid=grid,
        in_specs=[
            pl.BlockSpec((NC, R, D_EMB), lambda i: (0, i, 0)),
            pl.BlockSpec((NC, R, W1), lambda i: (0, i, 0)),
            pl.BlockSpec((D_EMB, D_HID), lambda i: (0, 0)), full((1, D_HID)),
            pl.BlockSpec((D_HID, D_IN), lambda i: (0, 0)), full((1, D_IN)),
        ],
        out_specs=pl.BlockSpec((R, D_IN), lambda i: (i, 0)),
        out_shape=jax.ShapeDtypeStruct((N_ST_NODES, D_IN), jnp.float32),
    )(agg2, agg1, wfd, bfd, wd, bd)


def kernel(sc_data, st_x, edge_index, W_fe, b_fe, W_e, b_e, W_fd, b_fd, W_d, b_d):
    ones_col = jnp.ones((N_ST_NODES, 1), jnp.float32)
    pad = jnp.zeros((N_ST_NODES, W1 - D_IN - 1), jnp.float32)
    table1 = jnp.concatenate([st_x, ones_col, pad], axis=1)
    z1 = jnp.zeros((N_ST_NODES, W1), jnp.float32)
    z2 = jnp.zeros((N_ST_NODES, D_EMB), jnp.float32)

    bfe = b_fe.reshape(1, D_HID)
    be = b_e.reshape(1, D_EMB)
    bfd = b_fd.reshape(1, D_HID)
    bd = b_d.reshape(1, D_IN)

    agg1 = _make_sc_agg(W1, 80, 3, 1, 2, 1)(table1, edge_index, z1)
    sc_emb, sc_rec = _tc_sc_branch(
        sc_data, W_fe, bfe, W_e, be, W_fd, bfd, W_d, bd)
    st_emb = _tc_st_encode(agg1, W_fe, bfe, W_e, be)
    agg2 = _make_sc_agg(D_EMB, 80, 6, 2, 4, 2)(st_emb, edge_index, z2)
    st_rec = _tc_st_decode(agg2, agg1, W_fd, bfd, W_d, bd)
    return (sc_emb, st_emb, sc_rec, st_rec)


# speedup vs baseline: 21.8344x; 1.0022x over previous
import functools

import jax
import jax.numpy as jnp
from jax import lax
from jax.experimental import pallas as pl
from jax.experimental.pallas import tpu as pltpu
from jax.experimental.pallas import tpu_sc as plsc

N_SC_NODES = 20000
N_ST_NODES = 10000
N_EDGES = 320000
D_IN = 128
D_HID = 512
D_EMB = 16

NC = 2
NS = 16
NW = NC * NS
W1 = 144
EPT = N_EDGES // NW
RPT = N_ST_NODES // NS

_MM = dict(preferred_element_type=jnp.float32)


def _dotb(x, w):
    return jnp.dot(x.astype(jnp.bfloat16), w.astype(jnp.bfloat16), **_MM)


def _elu(x):
    return jnp.where(x > 0, x, jnp.exp(jnp.minimum(x, 0.0)) - 1.0)


@functools.lru_cache(maxsize=None)
def _make_sc_agg(width, ch, nslot, gd, idd, sd):
    nchunk = EPT // ch
    tail = EPT - nchunk * ch
    mesh = plsc.VectorSubcoreMesh(
        core_axis_name="c", subcore_axis_name="s", num_cores=NC, num_subcores=NS
    )

    @functools.partial(
        pl.kernel,
        out_type=jax.ShapeDtypeStruct((NC, N_ST_NODES, width), jnp.float32),
        mesh=mesh,
        scratch_types=(
            [pltpu.VMEM((ch,), jnp.int32) for _ in range(2 * nslot)]
            + [pltpu.VMEM((ch, width), jnp.float32) for _ in range(nslot)]
            + [pltpu.VMEM((tail or 8,), jnp.int32) for _ in range(2)]
            + [pltpu.VMEM((tail or 8, width), jnp.float32)]
            + [pltpu.VMEM_SHARED((N_ST_NODES, width), jnp.float32)]
            + [pltpu.SemaphoreType.DMA for _ in range(4 * nslot)]
        ),
        compiler_params=pltpu.CompilerParams(use_tc_tiling_on_sc=False),
    )
    def agg_kernel(table, edges, zeros, out, *scr):
        S = scr[0:nslot]
        D = scr[nslot:2 * nslot]
        R = scr[2 * nslot:3 * nslot]
        ts, td, tr = scr[3 * nslot:3 * nslot + 3]
        shared = scr[3 * nslot + 3]
        sems = scr[3 * nslot + 4:]
        SI = sems[0:nslot]
        DI = sems[nslot:2 * nslot]
        G = sems[2 * nslot:3 * nslot]
        SS = sems[3 * nslot:4 * nslot]

        c = lax.axis_index("c")
        s = lax.axis_index("s")
        worker = c * NS + s
        rr0 = s * RPT
        pltpu.sync_copy(zeros, shared.at[pl.ds(rr0, RPT)])
        plsc.subcore_barrier()

        ebase = worker * EPT

        def eslice(j):
            return pl.ds(ebase + j * ch, ch)

        def issue_idx(j, h):
            pltpu.async_copy(edges.at[0, eslice(j)], S[h], SI[h])
            pltpu.async_copy(edges.at[1, eslice(j)], D[h], DI[h])

        def wait_idx(j, h):
            pltpu.make_async_copy(edges.at[0, eslice(j)], S[h], SI[h]).wait()
            pltpu.make_async_copy(edges.at[1, eslice(j)], D[h], DI[h]).wait()

        def issue_gather(j, h):
            wait_idx(j, h)
            pltpu.async_copy(table.at[S[h]], R[h], G[h])

        for j in range(idd):
            issue_idx(j, j % nslot)
        for j in range(gd):
            issue_gather(j, j % nslot)

        def halfstep(k, h):
            hg = (h + gd) % nslot
            hi = (h + idd) % nslot
            hs = (h - sd) % nslot

            @pl.when((k >= sd) & (k < nchunk + sd))
            def _():
                pltpu.make_async_copy(R[hs], shared.at[D[hs]], SS[hs]).wait()

            @pl.when(k + gd < nchunk)
            def _():
                issue_gather(k + gd, hg)

            @pl.when(k < nchunk)
            def _():
                pltpu.make_async_copy(table.at[S[h]], R[h], G[h]).wait()
                pltpu.async_copy(R[h], shared.at[D[h]], SS[h], add=True)

            @pl.when(k + idd < nchunk)
            def _():
                issue_idx(k + idd, hi)

        def step(t, carry):
            for hh in range(nslot):
                halfstep(nslot * t + hh, hh)
            return carry

        lax.fori_loop(0, (nchunk + sd + nslot - 1) // nslot, step, 0)

        if tail:
            tb = ebase + nchunk * ch
            pltpu.sync_copy(edges.at[0, pl.ds(tb, tail)], ts)
            pltpu.sync_copy(edges.at[1, pl.ds(tb, tail)], td)
            pltpu.async_copy(table.at[ts], tr, G[0]).wait()
            pltpu.sync_copy(tr, shared.at[td], add=True)

        plsc.subcore_barrier()
        pltpu.sync_copy(shared.at[pl.ds(rr0, RPT)], out.at[c, pl.ds(rr0, RPT)])

    return agg_kernel


def _tc_sc_branch(x, wfe, bfe, we, be, wfd, bfd, wd, bd):
    R = 400
    grid = (N_SC_NODES // R,)

    def body(x_r, wfe_r, bfe_r, we_r, be_r, wfd_r, bfd_r, wd_r, bd_r, emb_r, rec_r):
        h = _dotb(x_r[...], wfe_r[...]) + bfe_r[...]
        emb = _dotb(_elu(h), we_r[...]) + be_r[...]
        emb_r[...] = emb
        rh = _dotb(emb, wfd_r[...]) + bfd_r[...]
        rec_r[...] = _dotb(_elu(rh), wd_r[...]) + bd_r[...]

    full = lambda shape: pl.BlockSpec(shape, lambda i: (0, 0))
    return pl.pallas_call(
        body,
        grid=grid,
        in_specs=[
            pl.BlockSpec((R, D_IN), lambda i: (i, 0)),
            full((D_IN, D_HID)), full((1, D_HID)),
            full((D_HID, D_EMB)), full((1, D_EMB)),
            full((D_EMB, D_HID)), full((1, D_HID)),
            full((D_HID, D_IN)), full((1, D_IN)),
        ],
        out_specs=[
            pl.BlockSpec((R, D_EMB), lambda i: (i, 0)),
            pl.BlockSpec((R, D_IN), lambda i: (i, 0)),
        ],
        out_shape=[
            jax.ShapeDtypeStruct((N_SC_NODES, D_EMB), jnp.float32),
            jax.ShapeDtypeStruct((N_SC_NODES, D_IN), jnp.float32),
        ],
    )(x, wfe, bfe, we, be, wfd, bfd, wd, bd)


def _tc_st_encode(agg1, wfe, bfe, we, be):
    R = 400
    grid = (N_ST_NODES // R,)

    def body(agg_r, wfe_r, bfe_r, we_r, be_r, emb_r):
        a = agg_r[0] + agg_r[1]
        deg = a[:, D_IN:D_IN + 1]
        x = a[:, :D_IN]
        nx = x / jnp.maximum(deg, 1.0)
        m = jnp.minimum(deg, 1.0)
        h = _dotb(nx, wfe_r[...]) + m * bfe_r[...]
        emb_r[...] = _dotb(_elu(h), we_r[...]) + be_r[...]

    full = lambda shape: pl.BlockSpec(shape, lambda i: (0, 0))
    return pl.pallas_call(
        body,
        grid=grid,
        in_specs=[
            pl.BlockSpec((NC, R, W1), lambda i: (0, i, 0)),
            pl.BlockSpec((D_IN, D_HID), lambda i: (0, 0)), full((1, D_HID)),
            pl.BlockSpec((D_HID, D_EMB), lambda i: (0, 0)), full((1, D_EMB)),
        ],
        out_specs=pl.BlockSpec((R, D_EMB), lambda i: (i, 0)),
        out_shape=jax.ShapeDtypeStruct((N_ST_NODES, D_EMB), jnp.float32),
    )(agg1, wfe, bfe, we, be)


def _tc_st_decode(agg2, agg1, wfd, bfd, wd, bd):
    R = 400
    grid = (N_ST_NODES // R,)

    def body(agg2_r, agg1_r, wfd_r, bfd_r, wd_r, bd_r, rec_r):
        a2 = agg2_r[0] + agg2_r[1]
        deg = agg1_r[0, :, D_IN:D_IN + 1] + agg1_r[1, :, D_IN:D_IN + 1]
        nx = a2 / jnp.maximum(deg, 1.0)
        m = jnp.minimum(deg, 1.0)
        rh = _dotb(nx, wfd_r[...]) + m * bfd_r[...]
        rec_r[...] = _dotb(_elu(rh), wd_r[...]) + bd_r[...]

    full = lambda shape: pl.BlockSpec(shape, lambda i: (0, 0))
    return pl.pallas_call(
        body,
        grid=grid,
        in_specs=[
            pl.BlockSpec((NC, R, D_EMB), lambda i: (0, i, 0)),
            pl.BlockSpec((NC, R, W1), lambda i: (0, i, 0)),
            pl.BlockSpec((D_EMB, D_HID), lambda i: (0, 0)), full((1, D_HID)),
            pl.BlockSpec((D_HID, D_IN), lambda i: (0, 0)), full((1, D_IN)),
        ],
        out_specs=pl.BlockSpec((R, D_IN), lambda i: (i, 0)),
        out_shape=jax.ShapeDtypeStruct((N_ST_NODES, D_IN), jnp.float32),
    )(agg2, agg1, wfd, bfd, wd, bd)


def kernel(sc_data, st_x, edge_index, W_fe, b_fe, W_e, b_e, W_fd, b_fd, W_d, b_d):
    ones_col = jnp.ones((N_ST_NODES, 1), jnp.float32)
    pad = jnp.zeros((N_ST_NODES, W1 - D_IN - 1), jnp.float32)
    table1 = jnp.concatenate([st_x, ones_col, pad], axis=1)
    z1 = jnp.zeros((RPT, W1), jnp.float32)
    z2 = jnp.zeros((RPT, D_EMB), jnp.float32)

    bfe = b_fe.reshape(1, D_HID)
    be = b_e.reshape(1, D_EMB)
    bfd = b_fd.reshape(1, D_HID)
    bd = b_d.reshape(1, D_IN)

    agg1 = _make_sc_agg(W1, 80, 3, 1, 2, 1)(table1, edge_index, z1)
    sc_emb, sc_rec = _tc_sc_branch(
        sc_data, W_fe, bfe, W_e, be, W_fd, bfd, W_d, bd)
    st_emb = _tc_st_encode(agg1, W_fe, bfe, W_e, be)
    agg2 = _make_sc_agg(D_EMB, 80, 6, 2, 4, 2)(st_emb, edge_index, z2)
    st_rec = _tc_st_decode(agg2, agg1, W_fd, bfd, W_d, bd)
    return (sc_emb, st_emb, sc_rec, st_rec)


# speedup vs baseline: 21.9397x; 1.0048x over previous
import functools

import jax
import jax.numpy as jnp
from jax import lax
from jax.experimental import pallas as pl
from jax.experimental.pallas import tpu as pltpu
from jax.experimental.pallas import tpu_sc as plsc

N_SC_NODES = 20000
N_ST_NODES = 10000
N_EDGES = 320000
D_IN = 128
D_HID = 512
D_EMB = 16

NC = 2
NS = 16
NW = NC * NS
W1 = 144
EPT = N_EDGES // NW
RPT = N_ST_NODES // NS

_MM = dict(preferred_element_type=jnp.float32)


def _dotb(x, w):
    return jnp.dot(x.astype(jnp.bfloat16), w.astype(jnp.bfloat16), **_MM)


def _elu(x):
    return jnp.where(x > 0, x, jnp.exp(jnp.minimum(x, 0.0)) - 1.0)


@functools.lru_cache(maxsize=None)
def _make_sc_agg(width, ch, nslot, gd, idd, sd):
    nchunk = EPT // ch
    tail = EPT - nchunk * ch
    mesh = plsc.VectorSubcoreMesh(
        core_axis_name="c", subcore_axis_name="s", num_cores=NC, num_subcores=NS
    )

    @functools.partial(
        pl.kernel,
        out_type=jax.ShapeDtypeStruct((NC, N_ST_NODES, width), jnp.float32),
        mesh=mesh,
        scratch_types=(
            [pltpu.VMEM((ch,), jnp.int32) for _ in range(2 * nslot)]
            + [pltpu.VMEM((ch, width), jnp.float32) for _ in range(nslot)]
            + [pltpu.VMEM((tail or 8,), jnp.int32) for _ in range(2)]
            + [pltpu.VMEM((tail or 8, width), jnp.float32)]
            + [pltpu.VMEM_SHARED((N_ST_NODES, width), jnp.float32)]
            + [pltpu.SemaphoreType.DMA for _ in range(4 * nslot)]
        ),
        compiler_params=pltpu.CompilerParams(use_tc_tiling_on_sc=False),
    )
    def agg_kernel(table, edges, zeros, out, *scr):
        S = scr[0:nslot]
        D = scr[nslot:2 * nslot]
        R = scr[2 * nslot:3 * nslot]
        ts, td, tr = scr[3 * nslot:3 * nslot + 3]
        shared = scr[3 * nslot + 3]
        sems = scr[3 * nslot + 4:]
        SI = sems[0:nslot]
        DI = sems[nslot:2 * nslot]
        G = sems[2 * nslot:3 * nslot]
        SS = sems[3 * nslot:4 * nslot]

        c = lax.axis_index("c")
        s = lax.axis_index("s")
        worker = c * NS + s
        rr0 = s * RPT
        pltpu.sync_copy(zeros, shared.at[pl.ds(rr0, RPT)])
        plsc.subcore_barrier()

        ebase = worker * EPT

        def eslice(j):
            return pl.ds(ebase + j * ch, ch)

        def issue_idx(j, h):
            pltpu.async_copy(edges.at[0, eslice(j)], S[h], SI[h])
            pltpu.async_copy(edges.at[1, eslice(j)], D[h], DI[h])

        def wait_idx(j, h):
            pltpu.make_async_copy(edges.at[0, eslice(j)], S[h], SI[h]).wait()
            pltpu.make_async_copy(edges.at[1, eslice(j)], D[h], DI[h]).wait()

        def issue_gather(j, h):
            wait_idx(j, h)
            pltpu.async_copy(table.at[S[h]], R[h], G[h])

        for j in range(idd):
            issue_idx(j, j % nslot)
        for j in range(gd):
            issue_gather(j, j % nslot)

        def halfstep(k, h):
            hg = (h + gd) % nslot
            hi = (h + idd) % nslot
            hs = (h - sd) % nslot

            @pl.when((k >= sd) & (k < nchunk + sd))
            def _():
                pltpu.make_async_copy(R[hs], shared.at[D[hs]], SS[hs]).wait()

            @pl.when(k + gd < nchunk)
            def _():
                issue_gather(k + gd, hg)

            @pl.when(k < nchunk)
            def _():
                pltpu.make_async_copy(table.at[S[h]], R[h], G[h]).wait()
                pltpu.async_copy(R[h], shared.at[D[h]], SS[h], add=True)

            @pl.when(k + idd < nchunk)
            def _():
                issue_idx(k + idd, hi)

        def step(t, carry):
            for hh in range(nslot):
                halfstep(nslot * t + hh, hh)
            return carry

        lax.fori_loop(0, (nchunk + sd + nslot - 1) // nslot, step, 0)

        if tail:
            tb = ebase + nchunk * ch
            pltpu.sync_copy(edges.at[0, pl.ds(tb, tail)], ts)
            pltpu.sync_copy(edges.at[1, pl.ds(tb, tail)], td)
            pltpu.async_copy(table.at[ts], tr, G[0]).wait()
            pltpu.sync_copy(tr, shared.at[td], add=True)

        plsc.subcore_barrier()
        pltpu.sync_copy(shared.at[pl.ds(rr0, RPT)], out.at[c, pl.ds(rr0, RPT)])

    return agg_kernel


def _tc_sc_branch(x, wfe, bfe, we, be, wfd, bfd, wd, bd):
    R = 400
    grid = (N_SC_NODES // R,)

    def body(x_r, wfe_r, bfe_r, we_r, be_r, wfd_r, bfd_r, wd_r, bd_r, emb_r, rec_r):
        h = _dotb(x_r[...], wfe_r[...]) + bfe_r[...]
        emb = _dotb(_elu(h), we_r[...]) + be_r[...]
        emb_r[...] = emb
        rh = _dotb(emb, wfd_r[...]) + bfd_r[...]
        rec_r[...] = _dotb(_elu(rh), wd_r[...]) + bd_r[...]

    full = lambda shape: pl.BlockSpec(shape, lambda i: (0, 0))
    return pl.pallas_call(
        body,
        grid=grid,
        in_specs=[
            pl.BlockSpec((R, D_IN), lambda i: (i, 0)),
            full((D_IN, D_HID)), full((1, D_HID)),
            full((D_HID, D_EMB)), full((1, D_EMB)),
            full((D_EMB, D_HID)), full((1, D_HID)),
            full((D_HID, D_IN)), full((1, D_IN)),
        ],
        out_specs=[
            pl.BlockSpec((R, D_EMB), lambda i: (i, 0)),
            pl.BlockSpec((R, D_IN), lambda i: (i, 0)),
        ],
        out_shape=[
            jax.ShapeDtypeStruct((N_SC_NODES, D_EMB), jnp.float32),
            jax.ShapeDtypeStruct((N_SC_NODES, D_IN), jnp.float32),
        ],
    )(x, wfe, bfe, we, be, wfd, bfd, wd, bd)


def _tc_st_encode(agg1, wfe, bfe, we, be):
    R = 400
    grid = (N_ST_NODES // R,)

    def body(agg_r, wfe_r, bfe_r, we_r, be_r, emb_r, deg_r):
        a = agg_r[0] + agg_r[1]
        deg = a[:, D_IN:D_IN + 1]
        x = a[:, :D_IN]
        nx = x / jnp.maximum(deg, 1.0)
        m = jnp.minimum(deg, 1.0)
        h = _dotb(nx, wfe_r[...]) + m * bfe_r[...]
        emb_r[...] = _dotb(_elu(h), we_r[...]) + be_r[...]
        deg_r[...] = jnp.broadcast_to(deg, (R, 8))

    full = lambda shape: pl.BlockSpec(shape, lambda i: (0, 0))
    return pl.pallas_call(
        body,
        grid=grid,
        in_specs=[
            pl.BlockSpec((NC, R, W1), lambda i: (0, i, 0)),
            pl.BlockSpec((D_IN, D_HID), lambda i: (0, 0)), full((1, D_HID)),
            pl.BlockSpec((D_HID, D_EMB), lambda i: (0, 0)), full((1, D_EMB)),
        ],
        out_specs=[
            pl.BlockSpec((R, D_EMB), lambda i: (i, 0)),
            pl.BlockSpec((R, 8), lambda i: (i, 0)),
        ],
        out_shape=[
            jax.ShapeDtypeStruct((N_ST_NODES, D_EMB), jnp.float32),
            jax.ShapeDtypeStruct((N_ST_NODES, 8), jnp.float32),
        ],
    )(agg1, wfe, bfe, we, be)


def _tc_st_decode(agg2, degm, wfd, bfd, wd, bd):
    R = 400
    grid = (N_ST_NODES // R,)

    def body(agg2_r, deg_r, wfd_r, bfd_r, wd_r, bd_r, rec_r):
        a2 = agg2_r[0] + agg2_r[1]
        deg = deg_r[...][:, 0:1]
        nx = a2 / jnp.maximum(deg, 1.0)
        m = jnp.minimum(deg, 1.0)
        rh = _dotb(nx, wfd_r[...]) + m * bfd_r[...]
        rec_r[...] = _dotb(_elu(rh), wd_r[...]) + bd_r[...]

    full = lambda shape: pl.BlockSpec(shape, lambda i: (0, 0))
    return pl.pallas_call(
        body,
        grid=grid,
        in_specs=[
            pl.BlockSpec((NC, R, D_EMB), lambda i: (0, i, 0)),
            pl.BlockSpec((R, 8), lambda i: (i, 0)),
            pl.BlockSpec((D_EMB, D_HID), lambda i: (0, 0)), full((1, D_HID)),
            pl.BlockSpec((D_HID, D_IN), lambda i: (0, 0)), full((1, D_IN)),
        ],
        out_specs=pl.BlockSpec((R, D_IN), lambda i: (i, 0)),
        out_shape=jax.ShapeDtypeStruct((N_ST_NODES, D_IN), jnp.float32),
    )(agg2, degm, wfd, bfd, wd, bd)


def kernel(sc_data, st_x, edge_index, W_fe, b_fe, W_e, b_e, W_fd, b_fd, W_d, b_d):
    ones_col = jnp.ones((N_ST_NODES, 1), jnp.float32)
    pad = jnp.zeros((N_ST_NODES, W1 - D_IN - 1), jnp.float32)
    table1 = jnp.concatenate([st_x, ones_col, pad], axis=1)
    z1 = jnp.zeros((RPT, W1), jnp.float32)
    z2 = jnp.zeros((RPT, D_EMB), jnp.float32)

    bfe = b_fe.reshape(1, D_HID)
    be = b_e.reshape(1, D_EMB)
    bfd = b_fd.reshape(1, D_HID)
    bd = b_d.reshape(1, D_IN)

    agg1 = _make_sc_agg(W1, 80, 3, 1, 2, 1)(table1, edge_index, z1)
    sc_emb, sc_rec = _tc_sc_branch(
        sc_data, W_fe, bfe, W_e, be, W_fd, bfd, W_d, bd)
    st_emb, degm = _tc_st_encode(agg1, W_fe, bfe, W_e, be)
    agg2 = _make_sc_agg(D_EMB, 80, 6, 2, 4, 2)(st_emb, edge_index, z2)
    st_rec = _tc_st_decode(agg2, degm, W_fd, bfd, W_d, bd)
    return (sc_emb, st_emb, sc_rec, st_rec)
